# baseline (device time: 82151 ns/iter reference)
import numpy as np
import jax
import jax.numpy as jnp
from jax import lax
from jax.experimental import pallas as pl
from jax.experimental.pallas import tpu as pltpu

N_DEV = 8
B, SQ, D = 2, 512, 1024
HL, DH = 8, 128
SCALE = 0.08838834764831843


def _rope_consts():
    inv = 1.0 / (10000.0 ** (np.arange(0, DH, 2) / DH))
    pos = np.arange(SQ)[:, None] * inv[None, :]
    cos = np.repeat(np.cos(pos), 2, axis=-1).astype(np.float32)
    sin = np.repeat(np.sin(pos), 2, axis=-1).astype(np.float32)
    R = np.zeros((DH, DH), dtype=np.float32)
    idx = np.arange(0, DH, 2)
    R[idx + 1, idx] = -1.0
    R[idx, idx + 1] = 1.0
    return cos, sin, R


def _body(x_ref, wq_ref, wk_ref, wv_ref, wo_ref, cos_ref, sin_ref, r_ref,
          out_ref, ctx_ref, accb, recv_pool, send_sems, recv_sems):
    my = lax.axis_index("i")
    f32 = jnp.float32
    bf16 = jnp.bfloat16

    barrier_sem = pltpu.get_barrier_semaphore()
    for mask in (1, 3, 4):
        pl.semaphore_signal(barrier_sem, inc=1,
                            device_id=(jnp.bitwise_xor(my, mask),),
                            device_id_type=pl.DeviceIdType.MESH)
    pl.semaphore_wait(barrier_sem, 3)

    bit = {
        1: jnp.bitwise_xor(my & 1, (my >> 1) & 1),
        3: (my >> 1) & 1,
        4: (my >> 2) & 1,
    }
    halves = [
        {"start": 0, "rs": (1, 3, 4), "ag": (4, 3, 1), "pool": 0},
        {"start": 512, "rs": (3, 1, 4), "ag": (4, 1, 3), "pool": 448},
    ]
    rs_rows = (256, 128, 64)
    rs_off = (0, 256, 384)
    ag_rows = (64, 128, 256)

    def issue(idx, s):
        H = halves[idx]
        sem = 6 * idx + s
        if s < 3:
            n = rs_rows[s]
            mask = H["rs"][s]
            hi = bit[mask]
            keep = H["start"] + hi * n
            send_off = H["start"] + (1 - hi) * n
            po = H["pool"] + rs_off[s]
            rdma = pltpu.make_async_remote_copy(
                src_ref=accb.at[pl.ds(send_off, n)],
                dst_ref=recv_pool.at[pl.ds(po, n)],
                send_sem=send_sems.at[sem],
                recv_sem=recv_sems.at[sem],
                device_id=(jnp.bitwise_xor(my, mask),),
                device_id_type=pl.DeviceIdType.MESH,
            )
            rdma.start()
            H["start"] = keep
            return (rdma, keep, n, po)
        t = s - 3
        n = ag_rows[t]
        mask = H["ag"][t]
        hi = bit[mask]
        rdma = pltpu.make_async_remote_copy(
            src_ref=accb.at[pl.ds(H["start"], n)],
            dst_ref=accb.at[pl.ds(H["start"], n)],
            send_sem=send_sems.at[sem],
            recv_sem=recv_sems.at[sem],
            device_id=(jnp.bitwise_xor(my, mask),),
            device_id_type=pl.DeviceIdType.MESH,
        )
        rdma.start()
        H["start"] = H["start"] - hi * n
        return (rdma, None, n, None)

    def finish(pend):
        for rdma, keep, n, po in pend:
            rdma.wait()
        for rdma, keep, n, po in pend:
            if keep is not None:
                accb[pl.ds(keep, n), :] = (
                    accb[pl.ds(keep, n), :]
                    + recv_pool[pl.ds(po, n), :])

    xb = x_ref[...].astype(bf16)
    q_all = jnp.dot(xb, wq_ref[...].astype(bf16), preferred_element_type=f32)
    k_all = jnp.dot(xb, wk_ref[...].astype(bf16), preferred_element_type=f32)
    v_all = jnp.dot(xb, wv_ref[...].astype(bf16), preferred_element_type=f32)

    cos = cos_ref[...]
    sin = sin_ref[...]
    Rb = r_ref[...].astype(bf16)
    wo_b = wo_ref[...].astype(bf16)

    def attn_batch(b):
        rows = slice(b * SQ, (b + 1) * SQ)
        for h in range(HL):
            cols = slice(h * DH, (h + 1) * DH)
            q = q_all[rows, cols]
            k = k_all[rows, cols]
            q = q * cos + jnp.dot(q.astype(bf16), Rb,
                                  preferred_element_type=f32) * sin
            k = k * cos + jnp.dot(k.astype(bf16), Rb,
                                  preferred_element_type=f32) * sin
            s = lax.dot_general(
                q.astype(bf16), k.astype(bf16),
                (((1,), (1,)), ((), ())), preferred_element_type=f32) * SCALE
            m = jnp.max(s, axis=-1, keepdims=True)
            e = jnp.exp(s - m)
            den = jnp.sum(e, axis=-1, keepdims=True)
            w = (e / den).astype(bf16)
            v = v_all[rows, cols].astype(bf16)
            ctx_ref[rows, cols] = jnp.dot(
                w, v, preferred_element_type=f32).astype(bf16)

    attn_batch(0)
    accb[0:SQ, :] = jnp.dot(ctx_ref[0:SQ, :], wo_b,
                            preferred_element_type=f32).astype(bf16)
    pend = [issue(0, 0)]
    attn_batch(1)
    finish(pend)
    pend = [issue(0, 1)]
    accb[SQ:2 * SQ, :] = jnp.dot(ctx_ref[SQ:2 * SQ, :], wo_b,
                                 preferred_element_type=f32).astype(bf16)
    finish(pend)
    for j in range(6):
        pend = []
        if j + 2 <= 5:
            pend.append(issue(0, j + 2))
        pend.append(issue(1, j))
        finish(pend)

    out_ref[...] = accb[...].astype(f32)


def kernel(x, Wq, Wk, Wv, Wo):
    cos, sin, R = _rope_consts()
    xf = x.reshape(B * SQ, D)

    out = pl.pallas_call(
        _body,
        out_shape=jax.ShapeDtypeStruct((B * SQ, D), jnp.float32),
        in_specs=[pl.BlockSpec(memory_space=pltpu.VMEM)] * 8,
        out_specs=pl.BlockSpec(memory_space=pltpu.VMEM),
        scratch_shapes=[
            pltpu.VMEM((B * SQ, HL * DH), jnp.bfloat16),
            pltpu.VMEM((B * SQ, D), jnp.bfloat16),
            pltpu.VMEM((896, D), jnp.bfloat16),
            pltpu.SemaphoreType.DMA((12,)),
            pltpu.SemaphoreType.DMA((12,)),
        ],
        compiler_params=pltpu.CompilerParams(collective_id=0),
    )(xf, Wq, Wk, Wv, Wo, jnp.asarray(cos), jnp.asarray(sin), jnp.asarray(R))
    return out.reshape(B, SQ, D)


# device time: 72426 ns/iter; 1.1343x vs baseline; 1.1343x over previous
import numpy as np
import jax
import jax.numpy as jnp
from jax import lax
from jax.experimental import pallas as pl
from jax.experimental.pallas import tpu as pltpu

N_DEV = 8
B, SQ, D = 2, 512, 1024
HL, DH = 8, 128
SCALE = 0.08838834764831843


def _rope_consts():
    inv = 1.0 / (10000.0 ** (np.arange(0, DH, 2) / DH))
    pos = np.arange(SQ)[:, None] * inv[None, :]
    cos = np.repeat(np.cos(pos), 2, axis=-1).astype(np.float32)
    sin = np.repeat(np.sin(pos), 2, axis=-1).astype(np.float32)
    R = np.zeros((DH, DH), dtype=np.float32)
    idx = np.arange(0, DH, 2)
    R[idx + 1, idx] = -1.0
    R[idx, idx + 1] = 1.0
    return cos, sin, R


def _body(x_ref, wq_ref, wk_ref, wv_ref, wo_ref, cos_ref, sin_ref, r_ref,
          out_ref, ctx_ref, accb, recv_pool, send_sems, recv_sems):
    my = lax.axis_index("i")
    f32 = jnp.float32
    bf16 = jnp.bfloat16

    barrier_sem = pltpu.get_barrier_semaphore()
    for mask in (1, 3, 4):
        pl.semaphore_signal(barrier_sem, inc=1,
                            device_id=(jnp.bitwise_xor(my, mask),),
                            device_id_type=pl.DeviceIdType.MESH)
    pl.semaphore_wait(barrier_sem, 3)

    bit = {
        1: jnp.bitwise_xor(my & 1, (my >> 1) & 1),
        3: (my >> 1) & 1,
        4: (my >> 2) & 1,
    }
    halves = [
        {"start": 0, "rs": (1, 3, 4), "ag": (4, 3, 1), "pool": 0},
        {"start": 512, "rs": (3, 4, 1), "ag": (1, 4, 3), "pool": 448},
    ]
    rs_rows = (256, 128, 64)
    rs_off = (0, 256, 384)
    ag_rows = (64, 128, 256)

    def issue(idx, s):
        H = halves[idx]
        sem = 6 * idx + s
        if s < 3:
            n = rs_rows[s]
            mask = H["rs"][s]
            hi = bit[mask]
            keep = H["start"] + hi * n
            send_off = H["start"] + (1 - hi) * n
            po = H["pool"] + rs_off[s]
            rdma = pltpu.make_async_remote_copy(
                src_ref=accb.at[pl.ds(send_off, n)],
                dst_ref=recv_pool.at[pl.ds(po, n)],
                send_sem=send_sems.at[sem],
                recv_sem=recv_sems.at[sem],
                device_id=(jnp.bitwise_xor(my, mask),),
                device_id_type=pl.DeviceIdType.MESH,
            )
            rdma.start()
            H["start"] = keep
            return (rdma, keep, n, po)
        t = s - 3
        n = ag_rows[t]
        mask = H["ag"][t]
        hi = bit[mask]
        rdma = pltpu.make_async_remote_copy(
            src_ref=accb.at[pl.ds(H["start"], n)],
            dst_ref=accb.at[pl.ds(H["start"], n)],
            send_sem=send_sems.at[sem],
            recv_sem=recv_sems.at[sem],
            device_id=(jnp.bitwise_xor(my, mask),),
            device_id_type=pl.DeviceIdType.MESH,
        )
        rdma.start()
        H["start"] = H["start"] - hi * n
        return (rdma, None, n, None)

    def finish(pend):
        for rdma, keep, n, po in pend:
            rdma.wait()
        for rdma, keep, n, po in pend:
            if keep is not None:
                accb[pl.ds(keep, n), :] = (
                    accb[pl.ds(keep, n), :]
                    + recv_pool[pl.ds(po, n), :])

    xb = x_ref[...].astype(bf16)
    q_all = jnp.dot(xb, wq_ref[...].astype(bf16), preferred_element_type=f32)
    k_all = jnp.dot(xb, wk_ref[...].astype(bf16), preferred_element_type=f32)
    v_all = jnp.dot(xb, wv_ref[...].astype(bf16), preferred_element_type=f32)

    cos = cos_ref[...]
    sin = sin_ref[...]
    Rb = r_ref[...].astype(bf16)
    wo_b = wo_ref[...].astype(bf16)

    def attn_batch(b):
        rows = slice(b * SQ, (b + 1) * SQ)
        for h in range(HL):
            cols = slice(h * DH, (h + 1) * DH)
            q = q_all[rows, cols]
            k = k_all[rows, cols]
            q = q * cos + jnp.dot(q.astype(bf16), Rb,
                                  preferred_element_type=f32) * sin
            k = k * cos + jnp.dot(k.astype(bf16), Rb,
                                  preferred_element_type=f32) * sin
            s = lax.dot_general(
                q.astype(bf16), k.astype(bf16),
                (((1,), (1,)), ((), ())), preferred_element_type=f32) * SCALE
            m = jnp.max(s, axis=-1, keepdims=True)
            e = jnp.exp(s - m)
            den = jnp.sum(e, axis=-1, keepdims=True)
            w = (e / den).astype(bf16)
            v = v_all[rows, cols].astype(bf16)
            ctx_ref[rows, cols] = jnp.dot(
                w, v, preferred_element_type=f32).astype(bf16)

    attn_batch(0)
    attn_batch(1)
    accb[...] = jnp.dot(ctx_ref[...], wo_b,
                        preferred_element_type=f32).astype(bf16)
    for j in range(6):
        finish([issue(0, j), issue(1, j)])

    out_ref[...] = accb[...].astype(f32)


def kernel(x, Wq, Wk, Wv, Wo):
    cos, sin, R = _rope_consts()
    xf = x.reshape(B * SQ, D)

    out = pl.pallas_call(
        _body,
        out_shape=jax.ShapeDtypeStruct((B * SQ, D), jnp.float32),
        in_specs=[pl.BlockSpec(memory_space=pltpu.VMEM)] * 8,
        out_specs=pl.BlockSpec(memory_space=pltpu.VMEM),
        scratch_shapes=[
            pltpu.VMEM((B * SQ, HL * DH), jnp.bfloat16),
            pltpu.VMEM((B * SQ, D), jnp.bfloat16),
            pltpu.VMEM((896, D), jnp.bfloat16),
            pltpu.SemaphoreType.DMA((12,)),
            pltpu.SemaphoreType.DMA((12,)),
        ],
        compiler_params=pltpu.CompilerParams(collective_id=0),
    )(xf, Wq, Wk, Wv, Wo, jnp.asarray(cos), jnp.asarray(sin), jnp.asarray(R))
    return out.reshape(B, SQ, D)


# device time: 67010 ns/iter; 1.2260x vs baseline; 1.0808x over previous
import numpy as np
import jax
import jax.numpy as jnp
from jax import lax
from jax.experimental import pallas as pl
from jax.experimental.pallas import tpu as pltpu

N_DEV = 8
B, SQ, D = 2, 512, 1024
HL, DH = 8, 128
SCALE = 0.08838834764831843


def _rope_consts():
    inv = 1.0 / (10000.0 ** (np.arange(0, DH, 2) / DH))
    pos = np.arange(SQ)[:, None] * inv[None, :]
    cos = np.repeat(np.cos(pos), 2, axis=-1)
    sin = np.repeat(np.sin(pos), 2, axis=-1)
    C = np.tile(cos, (B, HL)).astype(np.float32)
    S = np.tile(sin, (B, HL))
    odd = (np.arange(HL * DH) % 2 == 1)[None, :]
    S_ODD = (S * odd).astype(np.float32)
    S_EVEN = (S * ~odd).astype(np.float32)
    return C, S_ODD, S_EVEN


def _body(x_ref, wq_ref, wk_ref, wv_ref, wo_ref, c_ref, so_ref, se_ref,
          out_ref, ctx_ref, accb, recv_pool, send_sems, recv_sems):
    my = lax.axis_index("i")
    f32 = jnp.float32
    bf16 = jnp.bfloat16

    barrier_sem = pltpu.get_barrier_semaphore()
    for mask in (1, 3, 4):
        pl.semaphore_signal(barrier_sem, inc=1,
                            device_id=(jnp.bitwise_xor(my, mask),),
                            device_id_type=pl.DeviceIdType.MESH)
    pl.semaphore_wait(barrier_sem, 3)

    bit = {
        1: jnp.bitwise_xor(my & 1, (my >> 1) & 1),
        3: (my >> 1) & 1,
        4: (my >> 2) & 1,
    }
    halves = [
        {"start": 0, "rs": (1, 3, 4), "ag": (4, 3, 1), "pool": 0},
        {"start": 512, "rs": (3, 4, 1), "ag": (1, 4, 3), "pool": 448},
    ]
    rs_rows = (256, 128, 64)
    rs_off = (0, 256, 384)
    ag_rows = (64, 128, 256)

    def issue(idx, s):
        H = halves[idx]
        sem = 6 * idx + s
        if s < 3:
            n = rs_rows[s]
            mask = H["rs"][s]
            hi = bit[mask]
            keep = H["start"] + hi * n
            send_off = H["start"] + (1 - hi) * n
            po = H["pool"] + rs_off[s]
            rdma = pltpu.make_async_remote_copy(
                src_ref=accb.at[pl.ds(send_off, n)],
                dst_ref=recv_pool.at[pl.ds(po, n)],
                send_sem=send_sems.at[sem],
                recv_sem=recv_sems.at[sem],
                device_id=(jnp.bitwise_xor(my, mask),),
                device_id_type=pl.DeviceIdType.MESH,
            )
            rdma.start()
            H["start"] = keep
            return (rdma, keep, n, po)
        t = s - 3
        n = ag_rows[t]
        mask = H["ag"][t]
        hi = bit[mask]
        rdma = pltpu.make_async_remote_copy(
            src_ref=accb.at[pl.ds(H["start"], n)],
            dst_ref=accb.at[pl.ds(H["start"], n)],
            send_sem=send_sems.at[sem],
            recv_sem=recv_sems.at[sem],
            device_id=(jnp.bitwise_xor(my, mask),),
            device_id_type=pl.DeviceIdType.MESH,
        )
        rdma.start()
        H["start"] = H["start"] - hi * n
        return (rdma, None, n, None)

    def finish_one(pend):
        rdma, keep, n, po = pend
        rdma.wait()
        if keep is not None:
            accb[pl.ds(keep, n), :] = (
                accb[pl.ds(keep, n), :]
                + recv_pool[pl.ds(po, n), :])

    xb = x_ref[...].astype(bf16)
    q_all = jnp.dot(xb, wq_ref[...].astype(bf16), preferred_element_type=f32)
    k_all = jnp.dot(xb, wk_ref[...].astype(bf16), preferred_element_type=f32)
    v_all = jnp.dot(xb, wv_ref[...].astype(bf16), preferred_element_type=f32)

    C = c_ref[...]
    S_ODD = so_ref[...]
    S_EVEN = se_ref[...]
    q_all = (q_all * C + pltpu.roll(q_all, 1, 1) * S_ODD
             - pltpu.roll(q_all, D - 1, 1) * S_EVEN)
    k_all = (k_all * C + pltpu.roll(k_all, 1, 1) * S_ODD
             - pltpu.roll(k_all, D - 1, 1) * S_EVEN)
    wo_b = wo_ref[...].astype(bf16)

    def attn_batch(b):
        rows = slice(b * SQ, (b + 1) * SQ)
        for h in range(HL):
            cols = slice(h * DH, (h + 1) * DH)
            q = q_all[rows, cols].astype(bf16)
            k = k_all[rows, cols].astype(bf16)
            s = lax.dot_general(
                q, k, (((1,), (1,)), ((), ())),
                preferred_element_type=f32) * SCALE
            e = jnp.exp(s)
            rden = 1.0 / jnp.sum(e, axis=-1, keepdims=True)
            v = v_all[rows, cols].astype(bf16)
            ctx = jnp.dot(e.astype(bf16), v, preferred_element_type=f32)
            ctx_ref[rows, cols] = (ctx * rden).astype(bf16)

    attn_batch(0)
    attn_batch(1)
    accb[...] = jnp.dot(ctx_ref[...], wo_b,
                        preferred_element_type=f32).astype(bf16)
    for j in range(6):
        pa = issue(0, j)
        pb = issue(1, j)
        finish_one(pa)
        finish_one(pb)

    out_ref[...] = accb[...].astype(f32)


def kernel(x, Wq, Wk, Wv, Wo):
    C, S_ODD, S_EVEN = _rope_consts()
    xf = x.reshape(B * SQ, D)

    out = pl.pallas_call(
        _body,
        out_shape=jax.ShapeDtypeStruct((B * SQ, D), jnp.float32),
        in_specs=[pl.BlockSpec(memory_space=pltpu.VMEM)] * 8,
        out_specs=pl.BlockSpec(memory_space=pltpu.VMEM),
        scratch_shapes=[
            pltpu.VMEM((B * SQ, HL * DH), jnp.bfloat16),
            pltpu.VMEM((B * SQ, D), jnp.bfloat16),
            pltpu.VMEM((896, D), jnp.bfloat16),
            pltpu.SemaphoreType.DMA((12,)),
            pltpu.SemaphoreType.DMA((12,)),
        ],
        compiler_params=pltpu.CompilerParams(
            collective_id=0, vmem_limit_bytes=100 * 1024 * 1024),
    )(xf, Wq, Wk, Wv, Wo,
      jnp.asarray(C), jnp.asarray(S_ODD), jnp.asarray(S_EVEN))
    return out.reshape(B, SQ, D)


# device time: 63055 ns/iter; 1.3028x vs baseline; 1.0627x over previous
import numpy as np
import jax
import jax.numpy as jnp
from jax import lax
from jax.experimental import pallas as pl
from jax.experimental.pallas import tpu as pltpu

N_DEV = 8
B, SQ, D = 2, 512, 1024
HL, DH = 8, 128
SCALE = 0.08838834764831843


def _rope_consts():
    inv = 1.0 / (10000.0 ** (np.arange(0, DH, 2) / DH))
    pos = np.arange(SQ)[:, None] * inv[None, :]
    cos = np.repeat(np.cos(pos), 2, axis=-1)
    sin = np.repeat(np.sin(pos), 2, axis=-1)
    C = np.tile(cos, (B, HL)).astype(np.float32)
    S = np.tile(sin, (B, HL))
    odd = (np.arange(HL * DH) % 2 == 1)[None, :]
    S_ODD = (S * odd).astype(np.float32)
    S_EVEN = (S * ~odd).astype(np.float32)
    return C, S_ODD, S_EVEN


def _body(x_ref, wq_ref, wk_ref, wv_ref, wo_ref, c_ref, so_ref, se_ref,
          out_ref, ctx_ref, accb, recv_pool, send_sems, recv_sems):
    my = lax.axis_index("i")
    f32 = jnp.float32
    bf16 = jnp.bfloat16

    barrier_sem = pltpu.get_barrier_semaphore()
    for mask in (1, 3, 4):
        pl.semaphore_signal(barrier_sem, inc=1,
                            device_id=(jnp.bitwise_xor(my, mask),),
                            device_id_type=pl.DeviceIdType.MESH)
    pl.semaphore_wait(barrier_sem, 3)

    bit = {
        1: jnp.bitwise_xor(my & 1, (my >> 1) & 1),
        3: (my >> 1) & 1,
        4: (my >> 2) & 1,
    }
    halves = [
        {"start": 0, "rows": 384, "rs": (1, 3, 4), "ag": (4, 3, 1),
         "pool": 0},
        {"start": 384, "rows": 384, "rs": (3, 4, 1), "ag": (1, 4, 3),
         "pool": 336},
        {"start": 768, "rows": 256, "rs": (4, 1, 3), "ag": (3, 1, 4),
         "pool": 672},
    ]

    def issue(idx, s):
        H = halves[idx]
        sem = 6 * idx + s
        if s < 3:
            n = H["rows"] >> (s + 1)
            mask = H["rs"][s]
            hi = bit[mask]
            keep = H["start"] + hi * n
            send_off = H["start"] + (1 - hi) * n
            po = H["pool"] + (0, H["rows"] // 2, 3 * H["rows"] // 4)[s]
            rdma = pltpu.make_async_remote_copy(
                src_ref=accb.at[pl.ds(send_off, n)],
                dst_ref=recv_pool.at[pl.ds(po, n)],
                send_sem=send_sems.at[sem],
                recv_sem=recv_sems.at[sem],
                device_id=(jnp.bitwise_xor(my, mask),),
                device_id_type=pl.DeviceIdType.MESH,
            )
            rdma.start()
            H["start"] = keep
            return (rdma, keep, n, po)
        t = s - 3
        n = H["rows"] >> (3 - t)
        mask = H["ag"][t]
        hi = bit[mask]
        rdma = pltpu.make_async_remote_copy(
            src_ref=accb.at[pl.ds(H["start"], n)],
            dst_ref=accb.at[pl.ds(H["start"], n)],
            send_sem=send_sems.at[sem],
            recv_sem=recv_sems.at[sem],
            device_id=(jnp.bitwise_xor(my, mask),),
            device_id_type=pl.DeviceIdType.MESH,
        )
        rdma.start()
        H["start"] = H["start"] - hi * n
        return (rdma, None, n, None)

    def finish_one(pend):
        rdma, keep, n, po = pend
        rdma.wait()
        if keep is not None:
            accb[pl.ds(keep, n), :] = (
                accb[pl.ds(keep, n), :]
                + recv_pool[pl.ds(po, n), :])

    xb = x_ref[...].astype(bf16)
    q_all = jnp.dot(xb, wq_ref[...].astype(bf16), preferred_element_type=f32)
    k_all = jnp.dot(xb, wk_ref[...].astype(bf16), preferred_element_type=f32)
    v_all = jnp.dot(xb, wv_ref[...].astype(bf16), preferred_element_type=f32)

    C = c_ref[...]
    S_ODD = so_ref[...]
    S_EVEN = se_ref[...]
    q_all = (q_all * C + pltpu.roll(q_all, 1, 1) * S_ODD
             - pltpu.roll(q_all, D - 1, 1) * S_EVEN)
    k_all = (k_all * C + pltpu.roll(k_all, 1, 1) * S_ODD
             - pltpu.roll(k_all, D - 1, 1) * S_EVEN)
    wo_b = wo_ref[...].astype(bf16)

    def attn_batch(b):
        rows = slice(b * SQ, (b + 1) * SQ)
        for h in range(HL):
            cols = slice(h * DH, (h + 1) * DH)
            q = q_all[rows, cols].astype(bf16)
            k = k_all[rows, cols].astype(bf16)
            s = lax.dot_general(
                q, k, (((1,), (1,)), ((), ())),
                preferred_element_type=f32) * SCALE
            e = jnp.exp(s)
            rden = 1.0 / jnp.sum(e, axis=-1, keepdims=True)
            v = v_all[rows, cols].astype(bf16)
            ctx = jnp.dot(e.astype(bf16), v, preferred_element_type=f32)
            ctx_ref[rows, cols] = (ctx * rden).astype(bf16)

    attn_batch(0)
    attn_batch(1)
    accb[...] = jnp.dot(ctx_ref[...], wo_b,
                        preferred_element_type=f32).astype(bf16)
    for j in range(6):
        pend = [issue(idx, j) for idx in range(3)]
        for p in pend:
            finish_one(p)

    out_ref[...] = accb[...].astype(f32)


def kernel(x, Wq, Wk, Wv, Wo):
    C, S_ODD, S_EVEN = _rope_consts()
    xf = x.reshape(B * SQ, D)

    out = pl.pallas_call(
        _body,
        out_shape=jax.ShapeDtypeStruct((B * SQ, D), jnp.float32),
        in_specs=[pl.BlockSpec(memory_space=pltpu.VMEM)] * 8,
        out_specs=pl.BlockSpec(memory_space=pltpu.VMEM),
        scratch_shapes=[
            pltpu.VMEM((B * SQ, HL * DH), jnp.bfloat16),
            pltpu.VMEM((B * SQ, D), jnp.bfloat16),
            pltpu.VMEM((896, D), jnp.bfloat16),
            pltpu.SemaphoreType.DMA((18,)),
            pltpu.SemaphoreType.DMA((18,)),
        ],
        compiler_params=pltpu.CompilerParams(
            collective_id=0, vmem_limit_bytes=100 * 1024 * 1024),
    )(xf, Wq, Wk, Wv, Wo,
      jnp.asarray(C), jnp.asarray(S_ODD), jnp.asarray(S_EVEN))
    return out.reshape(B, SQ, D)


# device time: 59817 ns/iter; 1.3734x vs baseline; 1.0541x over previous
import numpy as np
import jax
import jax.numpy as jnp
from jax import lax
from jax.experimental import pallas as pl
from jax.experimental.pallas import tpu as pltpu

N_DEV = 8
B, SQ, D = 2, 512, 1024
HL, DH = 8, 128
SCALE = 0.08838834764831843


def _rope_consts():
    inv = 1.0 / (10000.0 ** (np.arange(0, DH, 2) / DH))
    pos = np.arange(SQ)[:, None] * inv[None, :]
    cos = np.repeat(np.cos(pos), 2, axis=-1)
    sin = np.repeat(np.sin(pos), 2, axis=-1)
    C = np.tile(cos, (B, HL)).astype(np.float32)
    S = np.tile(sin, (B, HL))
    odd = (np.arange(HL * DH) % 2 == 1)[None, :]
    S_ODD = (S * odd).astype(np.float32)
    S_EVEN = (S * ~odd).astype(np.float32)
    return C, S_ODD, S_EVEN


def _body(x_ref, wq_ref, wk_ref, wv_ref, wo_ref, c_ref, so_ref, se_ref,
          out_ref, ctx_ref, recv_pool, send_sems, recv_sems):
    my = lax.axis_index("i")
    f32 = jnp.float32
    bf16 = jnp.bfloat16

    barrier_sem = pltpu.get_barrier_semaphore()
    for mask in (1, 3, 4):
        pl.semaphore_signal(barrier_sem, inc=1,
                            device_id=(jnp.bitwise_xor(my, mask),),
                            device_id_type=pl.DeviceIdType.MESH)
    pl.semaphore_wait(barrier_sem, 3)

    bit = {
        1: jnp.bitwise_xor(my & 1, (my >> 1) & 1),
        3: (my >> 1) & 1,
        4: (my >> 2) & 1,
    }
    halves = [
        {"start": 0, "rows": 384, "rs": (1, 3, 4), "ag": (4, 3, 1),
         "pool": 0},
        {"start": 384, "rows": 384, "rs": (3, 4, 1), "ag": (1, 4, 3),
         "pool": 336},
        {"start": 768, "rows": 256, "rs": (4, 1, 3), "ag": (3, 1, 4),
         "pool": 672},
    ]

    def issue(idx, s):
        H = halves[idx]
        sem = 6 * idx + s
        if s < 3:
            n = H["rows"] >> (s + 1)
            mask = H["rs"][s]
            hi = bit[mask]
            keep = H["start"] + hi * n
            send_off = H["start"] + (1 - hi) * n
            po = H["pool"] + (0, H["rows"] // 2, 3 * H["rows"] // 4)[s]
            rdma = pltpu.make_async_remote_copy(
                src_ref=out_ref.at[pl.ds(send_off, n)],
                dst_ref=recv_pool.at[pl.ds(po, n)],
                send_sem=send_sems.at[sem],
                recv_sem=recv_sems.at[sem],
                device_id=(jnp.bitwise_xor(my, mask),),
                device_id_type=pl.DeviceIdType.MESH,
            )
            rdma.start()
            H["start"] = keep
            return (rdma, keep, n, po)
        t = s - 3
        n = H["rows"] >> (3 - t)
        mask = H["ag"][t]
        hi = bit[mask]
        rdma = pltpu.make_async_remote_copy(
            src_ref=out_ref.at[pl.ds(H["start"], n)],
            dst_ref=out_ref.at[pl.ds(H["start"], n)],
            send_sem=send_sems.at[sem],
            recv_sem=recv_sems.at[sem],
            device_id=(jnp.bitwise_xor(my, mask),),
            device_id_type=pl.DeviceIdType.MESH,
        )
        rdma.start()
        H["start"] = H["start"] - hi * n
        return (rdma, None, n, None)

    def finish_one(pend):
        rdma, keep, n, po = pend
        rdma.wait()
        if keep is not None:
            out_ref[pl.ds(keep, n), :] = (
                out_ref[pl.ds(keep, n), :]
                + recv_pool[pl.ds(po, n), :])

    xb = x_ref[...].astype(bf16)
    q_all = jnp.dot(xb, wq_ref[...].astype(bf16), preferred_element_type=f32)
    k_all = jnp.dot(xb, wk_ref[...].astype(bf16), preferred_element_type=f32)
    v_all = jnp.dot(xb, wv_ref[...].astype(bf16),
                    preferred_element_type=f32).astype(bf16)

    C = c_ref[...]
    S_ODD = so_ref[...]
    S_EVEN = se_ref[...]
    q_all = (q_all * C + pltpu.roll(q_all, 1, 1) * S_ODD
             - pltpu.roll(q_all, D - 1, 1) * S_EVEN)
    k_all = (k_all * C + pltpu.roll(k_all, 1, 1) * S_ODD
             - pltpu.roll(k_all, D - 1, 1) * S_EVEN)
    wo_b = wo_ref[...].astype(bf16)

    def attn_batch(b):
        rows = slice(b * SQ, (b + 1) * SQ)
        for h in range(HL):
            cols = slice(h * DH, (h + 1) * DH)
            q = q_all[rows, cols].astype(bf16)
            k = k_all[rows, cols].astype(bf16)
            s = lax.dot_general(
                q, k, (((1,), (1,)), ((), ())),
                preferred_element_type=f32) * SCALE
            e = jnp.exp(s)
            rden = 1.0 / jnp.sum(e, axis=-1, keepdims=True)
            v = v_all[rows, cols]
            ctx = jnp.dot(e.astype(bf16), v, preferred_element_type=f32)
            ctx_ref[rows, cols] = (ctx * rden).astype(bf16)

    attn_batch(0)
    attn_batch(1)
    out_ref[...] = jnp.dot(ctx_ref[...], wo_b,
                        preferred_element_type=f32).astype(bf16)
    for j in range(6):
        pend = [issue(idx, j) for idx in range(3)]
        for p in pend:
            finish_one(p)

def kernel(x, Wq, Wk, Wv, Wo):
    C, S_ODD, S_EVEN = _rope_consts()
    xf = x.reshape(B * SQ, D)

    out = pl.pallas_call(
        _body,
        out_shape=jax.ShapeDtypeStruct((B * SQ, D), jnp.bfloat16),
        in_specs=[pl.BlockSpec(memory_space=pltpu.VMEM)] * 8,
        out_specs=pl.BlockSpec(memory_space=pltpu.VMEM),
        scratch_shapes=[
            pltpu.VMEM((B * SQ, HL * DH), jnp.bfloat16),
            pltpu.VMEM((896, D), jnp.bfloat16),
            pltpu.SemaphoreType.DMA((18,)),
            pltpu.SemaphoreType.DMA((18,)),
        ],
        compiler_params=pltpu.CompilerParams(
            collective_id=0, vmem_limit_bytes=100 * 1024 * 1024),
    )(xf, Wq, Wk, Wv, Wo,
      jnp.asarray(C), jnp.asarray(S_ODD), jnp.asarray(S_EVEN))
    return out.reshape(B, SQ, D)


# device time: 59645 ns/iter; 1.3773x vs baseline; 1.0029x over previous
import numpy as np
import jax
import jax.numpy as jnp
from jax import lax
from jax.experimental import pallas as pl
from jax.experimental.pallas import tpu as pltpu

N_DEV = 8
B, SQ, D = 2, 512, 1024
HL, DH = 8, 128
SCALE = 0.08838834764831843


def _rope_consts():
    inv = 1.0 / (10000.0 ** (np.arange(0, DH, 2) / DH))
    pos = np.arange(SQ)[:, None] * inv[None, :]
    cos = np.repeat(np.cos(pos), 2, axis=-1)
    sin = np.repeat(np.sin(pos), 2, axis=-1)
    C = np.tile(cos, (B, HL)).astype(np.float32)
    S = np.tile(sin, (B, HL))
    odd = (np.arange(HL * DH) % 2 == 1)[None, :]
    S_ODD = (S * odd).astype(np.float32)
    S_EVEN = (S * ~odd).astype(np.float32)
    return C, S_ODD, S_EVEN


def _body(x_ref, wq_ref, wk_ref, wv_ref, wo_ref, c_ref, so_ref, se_ref,
          out_ref, ctx_ref, recv_pool, send_sems, recv_sems):
    my = lax.axis_index("i")
    f32 = jnp.float32
    bf16 = jnp.bfloat16

    barrier_sem = pltpu.get_barrier_semaphore()
    for mask in (1, 3, 4):
        pl.semaphore_signal(barrier_sem, inc=1,
                            device_id=(jnp.bitwise_xor(my, mask),),
                            device_id_type=pl.DeviceIdType.MESH)
    pl.semaphore_wait(barrier_sem, 3)

    bit = {
        1: jnp.bitwise_xor(my & 1, (my >> 1) & 1),
        3: (my >> 1) & 1,
        4: (my >> 2) & 1,
    }
    halves = [
        {"start": 0, "rows": 384, "rs": (1, 3, 4), "ag": (4, 3, 1),
         "pool": 0},
        {"start": 384, "rows": 384, "rs": (3, 4, 1), "ag": (1, 4, 3),
         "pool": 336},
        {"start": 768, "rows": 256, "rs": (4, 1, 3), "ag": (3, 1, 4),
         "pool": 672},
    ]

    def issue(idx, s):
        H = halves[idx]
        sem = 6 * idx + s
        if s < 3:
            n = H["rows"] >> (s + 1)
            mask = H["rs"][s]
            hi = bit[mask]
            keep = H["start"] + hi * n
            send_off = H["start"] + (1 - hi) * n
            po = H["pool"] + (0, H["rows"] // 2, 3 * H["rows"] // 4)[s]
            rdma = pltpu.make_async_remote_copy(
                src_ref=out_ref.at[pl.ds(send_off, n)],
                dst_ref=recv_pool.at[pl.ds(po, n)],
                send_sem=send_sems.at[sem],
                recv_sem=recv_sems.at[sem],
                device_id=(jnp.bitwise_xor(my, mask),),
                device_id_type=pl.DeviceIdType.MESH,
            )
            rdma.start()
            H["start"] = keep
            return (rdma, keep, n, po)
        t = s - 3
        n = H["rows"] >> (3 - t)
        mask = H["ag"][t]
        hi = bit[mask]
        rdma = pltpu.make_async_remote_copy(
            src_ref=out_ref.at[pl.ds(H["start"], n)],
            dst_ref=out_ref.at[pl.ds(H["start"], n)],
            send_sem=send_sems.at[sem],
            recv_sem=recv_sems.at[sem],
            device_id=(jnp.bitwise_xor(my, mask),),
            device_id_type=pl.DeviceIdType.MESH,
        )
        rdma.start()
        H["start"] = H["start"] - hi * n
        return (rdma, None, n, None)

    def finish_one(pend):
        rdma, keep, n, po = pend
        rdma.wait()
        if keep is not None:
            out_ref[pl.ds(keep, n), :] = (
                out_ref[pl.ds(keep, n), :]
                + recv_pool[pl.ds(po, n), :])

    xb = x_ref[...].astype(bf16)
    q_all = jnp.dot(xb, wq_ref[...].astype(bf16), preferred_element_type=f32)
    k_all = jnp.dot(xb, wk_ref[...].astype(bf16), preferred_element_type=f32)
    v_all = jnp.dot(xb, wv_ref[...].astype(bf16),
                    preferred_element_type=f32).astype(bf16)

    C = c_ref[...]
    S_ODD = so_ref[...]
    S_EVEN = se_ref[...]
    q_all = (q_all * C + pltpu.roll(q_all, 1, 1) * S_ODD
             - pltpu.roll(q_all, D - 1, 1) * S_EVEN)
    k_all = (k_all * C + pltpu.roll(k_all, 1, 1) * S_ODD
             - pltpu.roll(k_all, D - 1, 1) * S_EVEN)
    wo_b = wo_ref[...].astype(bf16)

    def attn_batch(b):
        rows = slice(b * SQ, (b + 1) * SQ)
        for h in range(HL):
            cols = slice(h * DH, (h + 1) * DH)
            q = q_all[rows, cols].astype(bf16)
            k = k_all[rows, cols].astype(bf16)
            s = lax.dot_general(
                q, k, (((1,), (1,)), ((), ())),
                preferred_element_type=f32) * SCALE
            e = jnp.exp(s)
            rden = 1.0 / jnp.sum(e, axis=-1, keepdims=True)
            v = v_all[rows, cols]
            ctx = jnp.dot(e.astype(bf16), v, preferred_element_type=f32)
            ctx_ref[rows, cols] = (ctx * rden).astype(bf16)

    attn_batch(0)
    attn_batch(1)
    out_ref[...] = jnp.dot(ctx_ref[...], wo_b,
                        preferred_element_type=f32).astype(bf16)
    pend = [issue(idx, 0) for idx in range(3)]
    for j in range(6):
        nxt = []
        for idx in range(3):
            finish_one(pend[idx])
            if j < 5:
                nxt.append(issue(idx, j + 1))
        pend = nxt

def kernel(x, Wq, Wk, Wv, Wo):
    C, S_ODD, S_EVEN = _rope_consts()
    xf = x.reshape(B * SQ, D)

    out = pl.pallas_call(
        _body,
        out_shape=jax.ShapeDtypeStruct((B * SQ, D), jnp.bfloat16),
        in_specs=[pl.BlockSpec(memory_space=pltpu.VMEM)] * 8,
        out_specs=pl.BlockSpec(memory_space=pltpu.VMEM),
        scratch_shapes=[
            pltpu.VMEM((B * SQ, HL * DH), jnp.bfloat16),
            pltpu.VMEM((896, D), jnp.bfloat16),
            pltpu.SemaphoreType.DMA((18,)),
            pltpu.SemaphoreType.DMA((18,)),
        ],
        compiler_params=pltpu.CompilerParams(
            collective_id=0, vmem_limit_bytes=100 * 1024 * 1024),
    )(xf, Wq, Wk, Wv, Wo,
      jnp.asarray(C), jnp.asarray(S_ODD), jnp.asarray(S_EVEN))
    return out.reshape(B, SQ, D)
